# Initial kernel scaffold; baseline (speedup 1.0000x reference)
import jax
import jax.numpy as jnp
from jax.experimental import pallas as pl


def kernel(x, edge_index, edge_attr, batch, params):
    raise NotImplementedError("write your pallas kernel here")


# speedup vs baseline: 1.0502x; 1.0502x over previous
import functools
import math

import jax
import jax.numpy as jnp
from jax.experimental import pallas as pl
from jax.experimental.pallas import tpu as pltpu

H = 256
BM = 256


def _pad_rows(x, bm=BM):
    m = x.shape[0]
    mp = ((m + bm - 1) // bm) * bm
    if mp == m:
        return x, m
    return jnp.pad(x, ((0, mp - m),) + ((0, 0),) * (x.ndim - 1)), m


def _mm_kernel(x_ref, w_ref, b_ref, o_ref, *, act):
    y = jnp.dot(x_ref[...], w_ref[...], preferred_element_type=jnp.float32)
    y = y + b_ref[...]
    if act == "relu":
        y = jnp.maximum(y, 0.0)
    elif act == "logsoftmax":
        y = y - jnp.max(y, axis=1, keepdims=True)
        y = y - jnp.log(jnp.sum(jnp.exp(y), axis=1, keepdims=True))
    o_ref[...] = y


def _mm(x, w, b, act=None):
    xp, m = _pad_rows(x)
    mp, k = xp.shape
    n = w.shape[1]
    out = pl.pallas_call(
        functools.partial(_mm_kernel, act=act),
        grid=(mp // BM,),
        in_specs=[
            pl.BlockSpec((BM, k), lambda i: (i, 0)),
            pl.BlockSpec((k, n), lambda i: (0, 0)),
            pl.BlockSpec((1, n), lambda i: (0, 0)),
        ],
        out_specs=pl.BlockSpec((BM, n), lambda i: (i, 0)),
        out_shape=jax.ShapeDtypeStruct((mp, n), jnp.float32),
    )(xp, w, b.reshape(1, n))
    return out[:m]


def _mm2_kernel(a_ref, wa_ref, x_ref, wx_ref, b_ref, o_ref, *, act):
    y = jnp.dot(a_ref[...], wa_ref[...], preferred_element_type=jnp.float32)
    y = y + jnp.dot(x_ref[...], wx_ref[...], preferred_element_type=jnp.float32)
    y = y + b_ref[...]
    if act == "relu":
        y = jnp.maximum(y, 0.0)
    o_ref[...] = y


def _mm2(a, wa, x, wx, b, act=None):
    ap, m = _pad_rows(a)
    xp, _ = _pad_rows(x)
    mp, k = ap.shape
    n = wa.shape[1]
    out = pl.pallas_call(
        functools.partial(_mm2_kernel, act=act),
        grid=(mp // BM,),
        in_specs=[
            pl.BlockSpec((BM, k), lambda i: (i, 0)),
            pl.BlockSpec((k, n), lambda i: (0, 0)),
            pl.BlockSpec((BM, k), lambda i: (i, 0)),
            pl.BlockSpec((k, n), lambda i: (0, 0)),
            pl.BlockSpec((1, n), lambda i: (0, 0)),
        ],
        out_specs=pl.BlockSpec((BM, n), lambda i: (i, 0)),
        out_shape=jax.ShapeDtypeStruct((mp, n), jnp.float32),
    )(ap, wa, xp, wx, b.reshape(1, n))
    return out[:m]


def _agg(x, edge_index, edge_weight):
    src, dst = edge_index[0], edge_index[1]
    n = x.shape[0]
    msg = jnp.take(x, src, axis=0, mode="clip")
    if edge_weight is not None:
        msg = msg * edge_weight[:, None]
    dst_safe = jnp.where(dst < 0, n, dst)
    return jnp.zeros((n + 1, x.shape[1]), x.dtype).at[dst_safe].add(msg)[:n]


def _conv(x, edge_index, edge_weight, w_rel, b_rel, w_root, act="relu"):
    agg = _agg(x, edge_index, edge_weight)
    return _mm2(agg, w_rel, x, w_root, b_rel, act=act)


def _pool(x, edge_index, s):
    n = x.shape[0]
    k = n // 2
    perm = jnp.argsort(-s)[:k]
    xk = jnp.take(x, perm, axis=0) * jnp.tanh(jnp.take(s, perm))[:, None]
    new_idx = jnp.full((n,), -1, jnp.int32).at[perm].set(
        jnp.arange(k, dtype=jnp.int32))
    prev_valid = (edge_index[0] >= 0) & (edge_index[1] >= 0)
    row = jnp.take(new_idx, edge_index[0], mode="clip")
    col = jnp.take(new_idx, edge_index[1], mode="clip")
    valid = prev_valid & (row >= 0) & (col >= 0)
    neg = jnp.int32(-1)
    row = jnp.where(valid, row, neg)
    col = jnp.where(valid, col, neg)
    return xk, jnp.stack([row, col])


def _score(h, edge_index, wp_rel, bp_rel, wp_root):
    agg = _agg(h, edge_index, None)
    return agg @ wp_rel[:, 0] + bp_rel[0] + h @ wp_root[:, 0]


def _readout(x):
    mx = jnp.max(x, axis=0)
    mean = jnp.mean(x, axis=0)
    return jnp.concatenate([mx, mean])[None, :]


def kernel(x, edge_index, edge_attr, batch, params):
    p = params
    h0 = _mm(x, p["W_emb"], p["b_emb"])
    ei = edge_index

    h1 = _conv(h0, ei, None, p["W1_rel"], p["b1_rel"], p["W1_root"])
    x_local = _mm(h1, p["Wl1"], p["bl1"])
    s1 = _score(h1, ei, p["Wp1_rel"], p["bp1_rel"], p["Wp1_root"])
    x1p, ei = _pool(h1, ei, s1)
    r1 = _readout(x1p)

    h2 = _conv(x1p, ei, edge_attr, p["W2_rel"], p["b2_rel"], p["W2_root"])
    s2 = _score(h2, ei, p["Wp2_rel"], p["bp2_rel"], p["Wp2_root"])
    x2p, ei = _pool(h2, ei, s2)
    r2 = _readout(x2p)

    h3 = _conv(x2p, ei, None, p["W3_rel"], p["b3_rel"], p["W3_root"])
    s3 = _score(h3, ei, p["Wp3_rel"], p["bp3_rel"], p["Wp3_root"])
    x3p, ei = _pool(h3, ei, s3)
    r3 = _readout(x3p)

    h4 = _conv(x3p, ei, edge_attr, p["W4_rel"], p["b4_rel"], p["W4_root"])
    s4 = _score(h4, ei, p["Wp4_rel"], p["bp4_rel"], p["Wp4_root"])
    x4p, ei = _pool(h4, ei, s4)
    r4 = _readout(x4p)

    xg = r1 + r2 + r3 + r4
    c1 = (xg @ p["Wlin1"][H:] + p["blin1"])[0]
    hh = _mm(x_local, p["Wlin1"][:H], c1, act="relu")
    hh = _mm(hh, p["Wlin2"], p["blin2"], act="relu")
    return _mm(hh, p["Wlin3"], p["blin3"], act="logsoftmax")
